# baseline (speedup 1.0000x reference)
import functools
import math

import jax
import jax.numpy as jnp
from jax import lax
from jax.experimental import pallas as pl
from jax.experimental.pallas import tpu as pltpu
from jax.experimental.pallas import tpu_sc as plsc

EMB = 64
SCALE = math.sqrt(EMB)
LANES = 16
NBUF = 3
PAIR = 2 * EMB


@functools.cache
def _build(BATCH, SEQ, V):
    info = plsc.get_sparse_core_info()
    NC, NS = info.num_cores, info.num_subcores
    NW = NC * NS
    rows_w = BATCH // NW
    toks_w = rows_w * SEQ
    n_chunks = rows_w
    assert BATCH % NW == 0
    assert n_chunks % NBUF == 2 and n_chunks >= 2 * NBUF, n_chunks
    GA, GB = 128, SEQ - 128
    assert 0 < GB <= 128 and SEQ % 8 == 0
    NGRP = -(-SEQ // LANES)

    mesh = plsc.VectorSubcoreMesh(core_axis_name="c", subcore_axis_name="s")

    @functools.partial(
        pl.kernel,
        mesh=mesh,
        out_type=jax.ShapeDtypeStruct((BATCH, SEQ, PAIR), jnp.float32),
        scratch_types=[
            pltpu.VMEM((toks_w + LANES,), jnp.int32),
            pltpu.VMEM((toks_w + LANES,), jnp.int32),
            pltpu.VMEM((NBUF, SEQ, PAIR), jnp.float32),
            pltpu.SemaphoreType.DMA((NBUF,)),
            pltpu.SemaphoreType.DMA((NBUF,)),
        ],
        compiler_params=pltpu.CompilerParams(
            use_tc_tiling_on_sc=False, needs_layout_passes=False
        ),
    )
    def emb_kernel(tokens_hbm, table_hbm, out_hbm, idx_all, par_all, rows_v,
                   gsem, ssem):
        wid = lax.axis_index("s") * NC + lax.axis_index("c")
        brow0 = wid * rows_w
        lanes = lax.iota(jnp.int32, LANES)

        pltpu.sync_copy(
            tokens_hbm.at[pl.ds(wid * toks_w, toks_w)],
            idx_all.at[pl.ds(0, toks_w)],
        )

        def split_body(i, carry):
            sl = pl.ds(i * LANES, LANES)
            v = idx_all[sl]
            par_all[sl] = (v & 1) * EMB
            idx_all[sl] = lax.shift_right_logical(v, jnp.int32(1))
            return carry

        lax.fori_loop(0, toks_w // LANES, split_body, 0)

        def fire(g, b):
            f0 = g * SEQ
            pltpu.async_copy(
                table_hbm.at[idx_all.at[pl.ds(f0, GA)]],
                rows_v.at[b, pl.ds(0, GA)],
                gsem.at[b],
            )
            pltpu.async_copy(
                table_hbm.at[idx_all.at[pl.ds(f0 + GA, GB)]],
                rows_v.at[b, pl.ds(GA, GB)],
                gsem.at[b],
            )

        def wait_gather(b):
            pltpu.make_async_copy(
                table_hbm.at[pl.ds(0, SEQ)], rows_v.at[b], gsem.at[b]
            ).wait()

        def store(g, b):
            pltpu.async_copy(
                rows_v.at[b], out_hbm.at[brow0 + g], ssem.at[b]
            )

        def wait_store(b):
            pltpu.make_async_copy(
                out_hbm.at[0], rows_v.at[b], ssem.at[b]
            ).wait()

        def compact(g, b):
            f0 = g * SEQ
            b_vec = lanes * 0 + b
            for grp in range(NGRP):
                n0 = grp * LANES
                tok = jnp.int32(n0) + lanes
                par = par_all[pl.ds(f0 + n0, LANES)]
                full = (grp + 1) * LANES <= SEQ
                mask = None if full else lanes < jnp.int32(SEQ - n0)

                @plsc.parallel_loop(0, EMB, step=1, unroll=4)
                def _(e):
                    e_vec = tok * 0 + e
                    if full:
                        vals = plsc.load_gather(rows_v, [b_vec, tok, par + e])
                        plsc.store_scatter(
                            rows_v, [b_vec, tok, e_vec], vals * SCALE
                        )
                    else:
                        vals = plsc.load_gather(
                            rows_v, [b_vec, tok, par + e], mask=mask
                        )
                        plsc.store_scatter(
                            rows_v, [b_vec, tok, e_vec], vals * SCALE,
                            mask=mask,
                        )

        def step(g, b, do_wait_store, do_fire):
            b2 = (b + 2) % NBUF
            wait_gather(b)
            compact(g, b)
            store(g, b)
            if do_fire:
                if do_wait_store:
                    wait_store(b2)
                fire(g + 2, b2)

        fire(0, 0)
        fire(1, 1)
        step(0, 0, False, True)
        step(1, 1, True, True)
        step(2, 2, True, True)

        def body(p, carry):
            g0 = NBUF * p
            for b in range(NBUF):
                step(g0 + b, b, True, True)
            return carry

        lax.fori_loop(1, n_chunks // NBUF, body, 0)

        step(n_chunks - 2, 0, False, False)
        step(n_chunks - 1, 1, False, False)
        wait_store(2)
        wait_store(0)
        wait_store(1)

    return emb_kernel


def kernel(tokens, table):
    BATCH, SEQ = tokens.shape
    V = table.shape[0]
    tokens_flat = tokens.reshape(BATCH * SEQ)
    table_pairs = table.reshape(V // 2, PAIR)
    out = _build(BATCH, SEQ, V)(tokens_flat, table_pairs)
    return out[:, :, :EMB]


# speedup vs baseline: 1.6276x; 1.6276x over previous
import functools
import math

import jax
import jax.numpy as jnp
from jax import lax
from jax.experimental import pallas as pl
from jax.experimental.pallas import tpu as pltpu
from jax.experimental.pallas import tpu_sc as plsc

EMB = 64
SCALE = math.sqrt(EMB)
LANES = 16
RBUF = 2
OBUF = 3
WIDE = 2 * EMB


@functools.cache
def _build(BATCH, SEQ, V):
    info = plsc.get_sparse_core_info()
    NC, NS = info.num_cores, info.num_subcores
    NW = NC * NS
    rows_w = BATCH // NW
    toks_w = rows_w * SEQ
    n_chunks = rows_w
    assert BATCH % NW == 0
    assert n_chunks == 128, n_chunks
    GA, GB = 128, SEQ - 128
    assert 0 < GB <= 128 and SEQ % 8 == 0

    mesh = plsc.VectorSubcoreMesh(core_axis_name="c", subcore_axis_name="s")

    @functools.partial(
        pl.kernel,
        mesh=mesh,
        out_type=jax.ShapeDtypeStruct((BATCH, SEQ, WIDE), jnp.float32),
        scratch_types=[
            pltpu.VMEM((toks_w,), jnp.int32),
            pltpu.VMEM((RBUF, SEQ, EMB), jnp.float32),
            pltpu.VMEM((OBUF, SEQ, WIDE), jnp.float32),
            pltpu.SemaphoreType.DMA((RBUF,)),
            pltpu.SemaphoreType.DMA((OBUF,)),
        ],
        compiler_params=pltpu.CompilerParams(use_tc_tiling_on_sc=False),
    )
    def emb_kernel(tokens_hbm, table_hbm, out_hbm, idx_all, rows_v, out_v,
                   gsem, ssem):
        wid = lax.axis_index("s") * NC + lax.axis_index("c")
        brow0 = wid * rows_w

        pltpu.sync_copy(tokens_hbm.at[pl.ds(wid * toks_w, toks_w)], idx_all)

        def fire(g, r):
            f0 = g * SEQ
            pltpu.async_copy(
                table_hbm.at[idx_all.at[pl.ds(f0, GA)]],
                rows_v.at[r, pl.ds(0, GA)],
                gsem.at[r],
            )
            pltpu.async_copy(
                table_hbm.at[idx_all.at[pl.ds(f0 + GA, GB)]],
                rows_v.at[r, pl.ds(GA, GB)],
                gsem.at[r],
            )

        def wait_gather(r):
            pltpu.make_async_copy(
                table_hbm.at[pl.ds(0, SEQ)], rows_v.at[r], gsem.at[r]
            ).wait()

        def store(g, o):
            pltpu.async_copy(
                out_v.at[o], out_hbm.at[brow0 + g], ssem.at[o]
            )

        def wait_store(o):
            pltpu.make_async_copy(
                out_hbm.at[0], out_v.at[o], ssem.at[o]
            ).wait()

        def scale_copy(r, o):
            rows_r = rows_v.at[r]
            out_o = out_v.at[o]

            @plsc.parallel_loop(0, SEQ, step=1, unroll=8)
            def _(n):
                for c in range(EMB // LANES):
                    sl = pl.ds(c * LANES, LANES)
                    out_o[n, sl] = rows_r[n, sl] * SCALE

        def step(g, r, o, do_wait_store, do_fire):
            wait_gather(r)
            if do_wait_store:
                wait_store(o)
            scale_copy(r, o)
            if do_fire:
                fire(g + 1, (r + 1) % RBUF)
            store(g, o)

        fire(0, 0)
        step(0, 0, 0, False, True)
        step(1, 1, 1, False, True)
        step(2, 0, 2, False, True)

        def body(p, carry):
            g0 = 3 + 6 * p
            for k in range(6):
                g = g0 + k
                step(g, (3 + k) % RBUF, (3 + k) % OBUF, True, True)
            return carry

        lax.fori_loop(0, 20, body, 0)

        step(123, 1, 0, True, True)
        step(124, 0, 1, True, True)
        step(125, 1, 2, True, True)
        step(126, 0, 0, True, True)
        step(127, 1, 1, True, False)
        wait_store(2)
        wait_store(0)
        wait_store(1)

    return emb_kernel


def kernel(tokens, table):
    BATCH, SEQ = tokens.shape
    V = table.shape[0]
    tokens_flat = tokens.reshape(BATCH * SEQ)
    out = _build(BATCH, SEQ, V)(tokens_flat, table)
    return out[:, :, :EMB]


# speedup vs baseline: 1.8108x; 1.1125x over previous
import functools
import math

import jax
import jax.numpy as jnp
from jax import lax
from jax.experimental import pallas as pl
from jax.experimental.pallas import tpu as pltpu
from jax.experimental.pallas import tpu_sc as plsc

EMB = 64
SCALE = math.sqrt(EMB)
LANES = 16
NBUF = 3
WIDE = 2 * EMB


@functools.cache
def _build(BATCH, SEQ, V):
    info = plsc.get_sparse_core_info()
    NC, NS = info.num_cores, info.num_subcores
    NW = NC * NS
    rows_w = BATCH // NW
    toks_w = rows_w * SEQ
    n_chunks = rows_w
    assert BATCH % NW == 0
    assert n_chunks % NBUF == 2 and n_chunks >= 2 * NBUF, n_chunks
    GA, GB = 128, SEQ - 128
    assert 0 < GB <= 128 and SEQ % 8 == 0

    mesh = plsc.VectorSubcoreMesh(core_axis_name="c", subcore_axis_name="s")

    @functools.partial(
        pl.kernel,
        mesh=mesh,
        out_type=jax.ShapeDtypeStruct((BATCH, SEQ, WIDE), jnp.float32),
        scratch_types=[
            pltpu.VMEM((toks_w,), jnp.int32),
            pltpu.VMEM((NBUF, SEQ, EMB), jnp.float32),
            pltpu.SemaphoreType.DMA((NBUF,)),
            pltpu.SemaphoreType.DMA((NBUF,)),
        ],
        compiler_params=pltpu.CompilerParams(use_tc_tiling_on_sc=False),
    )
    def emb_kernel(tokens_hbm, table_hbm, out_hbm, idx_all, rows_v, gsem, ssem):
        wid = lax.axis_index("s") * NC + lax.axis_index("c")
        brow0 = wid * rows_w

        pltpu.sync_copy(tokens_hbm.at[pl.ds(wid * toks_w, toks_w)], idx_all)

        def fire(g, b):
            f0 = g * SEQ
            pltpu.async_copy(
                table_hbm.at[idx_all.at[pl.ds(f0, GA)]],
                rows_v.at[b, pl.ds(0, GA)],
                gsem.at[b],
            )
            pltpu.async_copy(
                table_hbm.at[idx_all.at[pl.ds(f0 + GA, GB)]],
                rows_v.at[b, pl.ds(GA, GB)],
                gsem.at[b],
            )

        def wait_gather(b):
            pltpu.make_async_copy(
                table_hbm.at[pl.ds(0, SEQ)], rows_v.at[b], gsem.at[b]
            ).wait()

        def store(g, b):
            pltpu.async_copy(
                rows_v.at[b],
                out_hbm.at[brow0 + g, :, pl.ds(0, EMB)],
                ssem.at[b],
            )

        def wait_store(b):
            pltpu.make_async_copy(
                out_hbm.at[0, :, pl.ds(0, EMB)], rows_v.at[b], ssem.at[b]
            ).wait()

        def scale(b):
            rows_b = rows_v.at[b]

            @plsc.parallel_loop(0, SEQ, step=1, unroll=8)
            def _(n):
                for c in range(EMB // LANES):
                    sl = pl.ds(c * LANES, LANES)
                    rows_b[n, sl] = rows_b[n, sl] * SCALE

        def step(g, b, do_wait_store, do_fire):
            b2 = (b + 2) % NBUF
            wait_gather(b)
            scale(b)
            store(g, b)
            if do_fire:
                if do_wait_store:
                    wait_store(b2)
                fire(g + 2, b2)

        fire(0, 0)
        fire(1, 1)
        step(0, 0, False, True)
        step(1, 1, True, True)
        step(2, 2, True, True)

        def body(p, carry):
            g0 = NBUF * p
            for b in range(NBUF):
                step(g0 + b, b, True, True)
            return carry

        lax.fori_loop(1, n_chunks // NBUF, body, 0)

        step(n_chunks - 2, 0, False, False)
        step(n_chunks - 1, 1, False, False)
        wait_store(2)
        wait_store(0)
        wait_store(1)

    return emb_kernel


def kernel(tokens, table):
    BATCH, SEQ = tokens.shape
    V = table.shape[0]
    tokens_flat = tokens.reshape(BATCH * SEQ)
    out = _build(BATCH, SEQ, V)(tokens_flat, table)
    return out[:, :, :EMB]
